# Initial kernel scaffold; baseline (speedup 1.0000x reference)
import functools

import jax
import jax.numpy as jnp
from jax import lax
from jax.experimental import pallas as pl
from jax.experimental.pallas import tpu as pltpu
from jax.experimental.pallas import tpu_sc as plsc

N = 10000
E = 320000
D = 128
AUG = 144
L = 16
NC = 2
NS = 16
NW = NC * NS
PER_W = E // NW
CH = 80
NCH = PER_W // CH
ZR = 125
BLKN = 200


def _node_transform_body(x_ref, we_ref, web_ref, ws_ref, wsb_ref, wr_ref,
                         wrb_ref, aug_ref, a_ref, b_ref):
    xb = x_ref[...]
    xwe = lax.dot_general(xb, we_ref[...], (((1,), (0,)), ((), ())),
                          precision=lax.Precision.HIGHEST,
                          preferred_element_type=jnp.float32) + web_ref[...]
    a_ref[...] = lax.dot_general(xwe, ws_ref[...], (((1,), (0,)), ((), ())),
                                 precision=lax.Precision.HIGHEST,
                                 preferred_element_type=jnp.float32) + wsb_ref[...]
    b_ref[...] = lax.dot_general(xb, wr_ref[...], (((1,), (0,)), ((), ())),
                                 precision=lax.Precision.HIGHEST,
                                 preferred_element_type=jnp.float32) + wrb_ref[...]
    ones = jnp.ones((BLKN, 1), jnp.float32)
    zeros = jnp.zeros((BLKN, AUG - D - 1), jnp.float32)
    aug_ref[...] = jnp.concatenate([xwe, ones, zeros], axis=1)


def _node_transform(x, we, web, ws, wsb, wr, wrb):
    wspec = pl.BlockSpec((D, D), lambda i: (0, 0))
    bspec = pl.BlockSpec((1, D), lambda i: (0, 0))
    return pl.pallas_call(
        _node_transform_body,
        grid=(N // BLKN,),
        in_specs=[pl.BlockSpec((BLKN, D), lambda i: (i, 0)),
                  wspec, bspec, wspec, bspec, wspec, bspec],
        out_specs=[pl.BlockSpec((BLKN, AUG), lambda i: (i, 0)),
                   pl.BlockSpec((BLKN, D), lambda i: (i, 0)),
                   pl.BlockSpec((BLKN, D), lambda i: (i, 0))],
        out_shape=[jax.ShapeDtypeStruct((N, AUG), jnp.float32),
                   jax.ShapeDtypeStruct((N, D), jnp.float32),
                   jax.ShapeDtypeStruct((N, D), jnp.float32)],
    )(x, we, web, ws, wsb, wr, wrb)


def _sc_logits_body(a_hbm, b_hbm, s_hbm, r_hbm, wa_hbm, logits_hbm, tmax_hbm,
                    sidx, ridx, arows, brows, lbuf, wav, mbuf, sem_a, sem_b):
    c = lax.axis_index("c")
    s = lax.axis_index("s")
    wid = c * NS + s
    base = wid * PER_W
    pltpu.sync_copy(wa_hbm, wav)

    def chunk(j, mvec):
        off = base + j * CH
        pltpu.sync_copy(s_hbm.at[pl.ds(off, CH)], sidx.at[0])
        pltpu.sync_copy(r_hbm.at[pl.ds(off, CH)], ridx.at[0])
        cp_a = pltpu.async_copy(a_hbm.at[sidx.at[0]], arows, sem_a)
        cp_b = pltpu.async_copy(b_hbm.at[ridx.at[0]], brows, sem_b)
        cp_a.wait()
        cp_b.wait()

        @pl.loop(0, CH)
        def _edge(e):
            acc = jnp.zeros((L,), jnp.float32)
            for k in range(D // L):
                sl = pl.ds(k * L, L)
                t = arows[e, sl] + brows[e, sl]
                u = 1.0 + jnp.exp(jnp.minimum(t, 30.0))
                u2 = u * u
                z = t * ((u2 - 1.0) / (u2 + 1.0))
                acc = acc + z * wav[sl]
            lbuf[e] = jnp.sum(acc)

        mv = mvec
        for i in range(CH // L):
            mv = jnp.maximum(mv, lbuf[pl.ds(i * L, L)])
        pltpu.sync_copy(lbuf, logits_hbm.at[pl.ds(off, CH)])
        return mv

    mvec = lax.fori_loop(0, NCH, chunk, jnp.full((L,), -jnp.inf, jnp.float32))
    mbuf[...] = mvec
    pltpu.sync_copy(mbuf, tmax_hbm.at[wid])


def _sc_accum_body(logits_hbm, tmax_hbm, s_hbm, r_hbm, aug_hbm, out_hbm,
                   sidx, ridx, rows, lbuf, ubuf, tmv, zbuf, shared, sem):
    c = lax.axis_index("c")
    s = lax.axis_index("s")
    wid = c * NS + s
    base = wid * PER_W
    rows_per_tile = N // NS
    row0 = s * rows_per_tile

    @pl.loop(0, ZR)
    def _zero(i):
        for k in range(AUG // L):
            zbuf[i, pl.ds(k * L, L)] = jnp.zeros((L,), jnp.float32)

    for m in range(rows_per_tile // ZR):
        pltpu.sync_copy(zbuf, shared.at[pl.ds(row0 + m * ZR, ZR)])
    plsc.subcore_barrier()

    pltpu.sync_copy(tmax_hbm, tmv)
    mv = jnp.full((L,), -jnp.inf, jnp.float32)
    for i in range(NW):
        mv = jnp.maximum(mv, tmv[i])
    gmax = jnp.max(mv)

    @pl.loop(0, NCH)
    def _chunk(j):
        off = base + j * CH
        pltpu.sync_copy(s_hbm.at[pl.ds(off, CH)], sidx.at[0])
        pltpu.sync_copy(r_hbm.at[pl.ds(off, CH)], ridx.at[0])
        pltpu.sync_copy(logits_hbm.at[pl.ds(off, CH)], lbuf)
        pltpu.async_copy(aug_hbm.at[sidx.at[0]], rows, sem).wait()
        for i in range(CH // L):
            sl = pl.ds(i * L, L)
            ubuf[sl] = jnp.exp(lbuf[sl] - gmax)

        @pl.loop(0, CH)
        def _edge(e):
            ub = lax.broadcast_in_dim(ubuf[e], (L,), ())
            for k in range(AUG // L):
                sl = pl.ds(k * L, L)
                rows[e, sl] = rows[e, sl] * ub

        pltpu.sync_copy(rows, shared.at[ridx.at[0]], add=True)

    plsc.subcore_barrier()
    pltpu.sync_copy(shared.at[pl.ds(row0, rows_per_tile)],
                    out_hbm.at[c].at[pl.ds(row0, rows_per_tile)])


def _combine_body(p_ref, o_ref):
    p = p_ref[...]
    ssum = p[0] + p[1]
    den = ssum[:, D:D + 1]
    num = ssum[:, :D]
    o_ref[...] = jnp.where(den > 0.0, num / den, 0.0)


def _combine(partials):
    return pl.pallas_call(
        _combine_body,
        grid=(N // BLKN,),
        in_specs=[pl.BlockSpec((NC, BLKN, AUG), lambda i: (0, i, 0))],
        out_specs=pl.BlockSpec((BLKN, D), lambda i: (i, 0)),
        out_shape=jax.ShapeDtypeStruct((N, D), jnp.float32),
    )(partials)


def kernel(x, senders, receivers, We_k, We_b, Ws_k, Ws_b, Wr_k, Wr_b, Wa_k,
           Wa_b):
    s32 = senders.astype(jnp.int32)
    r32 = receivers.astype(jnp.int32)
    we = We_k.reshape(D, D)
    wr = Wr_k.reshape(D, D)
    web = We_b.reshape(1, D)
    wrb = Wr_b.reshape(1, D)
    wsb = Ws_b.reshape(1, D)
    wa = Wa_k.reshape(D)

    aug, a, b = _node_transform(x, we, web, Ws_k, wsb, wr, wrb)

    mesh = plsc.VectorSubcoreMesh(core_axis_name="c", subcore_axis_name="s")

    logits_fn = pl.kernel(
        _sc_logits_body,
        out_type=[jax.ShapeDtypeStruct((E,), jnp.float32),
                  jax.ShapeDtypeStruct((NW, L), jnp.float32)],
        mesh=mesh,
        scratch_types=[
            pltpu.VMEM((1, CH), jnp.int32),
            pltpu.VMEM((1, CH), jnp.int32),
            pltpu.VMEM((CH, D), jnp.float32),
            pltpu.VMEM((CH, D), jnp.float32),
            pltpu.VMEM((CH,), jnp.float32),
            pltpu.VMEM((D,), jnp.float32),
            pltpu.VMEM((L,), jnp.float32),
            pltpu.SemaphoreType.DMA,
            pltpu.SemaphoreType.DMA,
        ],
    )
    logits, tmax = logits_fn(a, b, s32, r32, wa)

    accum_fn = pl.kernel(
        _sc_accum_body,
        out_type=jax.ShapeDtypeStruct((NC, N, AUG), jnp.float32),
        mesh=mesh,
        scratch_types=[
            pltpu.VMEM((1, CH), jnp.int32),
            pltpu.VMEM((1, CH), jnp.int32),
            pltpu.VMEM((CH, AUG), jnp.float32),
            pltpu.VMEM((CH,), jnp.float32),
            pltpu.VMEM((CH,), jnp.float32),
            pltpu.VMEM((NW, L), jnp.float32),
            pltpu.VMEM((ZR, AUG), jnp.float32),
            pltpu.VMEM_SHARED((N, AUG), jnp.float32),
            pltpu.SemaphoreType.DMA,
        ],
    )
    partials = accum_fn(logits, tmax, s32, r32, aug)

    return _combine(partials)


# speedup vs baseline: 4.7071x; 4.7071x over previous
import dataclasses
import functools

import jax
import jax.numpy as jnp
from jax import lax
from jax.experimental import pallas as pl
from jax.experimental.pallas import tpu as pltpu
from jax.experimental.pallas import tpu_sc as plsc

N = 10000
E = 320000
D = 128
L = 16
NC = 2
NS = 16
NW = NC * NS
PER_W = E // NW
CH = 80
NCH = PER_W // CH
PER_W2 = E // NS
NCH2 = PER_W2 // CH
NPAD = 10240
SHALF = NPAD // NC
SROWS = 5248
RPT2 = SROWS // NS
ZR = 8
NDEN = 4
BLKN = 200
BLKC = 256
BLKO = 80
XROWS = 20400


def _node_transform_body(x_ref, we_ref, web_ref, ws_ref, wsb_ref, wr_ref,
                         wrb_ref, xwe_ref, a_ref, b_ref):
    xb = x_ref[...]
    xwe = lax.dot_general(xb, we_ref[...], (((1,), (0,)), ((), ())),
                          precision=lax.Precision.HIGHEST,
                          preferred_element_type=jnp.float32) + web_ref[...]
    xwe_ref[...] = xwe
    a_ref[...] = lax.dot_general(xwe, ws_ref[...], (((1,), (0,)), ((), ())),
                                 precision=lax.Precision.HIGHEST,
                                 preferred_element_type=jnp.float32) + wsb_ref[...]
    b_ref[...] = lax.dot_general(xb, wr_ref[...], (((1,), (0,)), ((), ())),
                                 precision=lax.Precision.HIGHEST,
                                 preferred_element_type=jnp.float32) + wrb_ref[...]


def _node_transform(x, we, web, ws, wsb, wr, wrb):
    wspec = pl.BlockSpec((D, D), lambda i: (0, 0))
    bspec = pl.BlockSpec((1, D), lambda i: (0, 0))
    nspec = pl.BlockSpec((BLKN, D), lambda i: (i, 0))
    return pl.pallas_call(
        _node_transform_body,
        grid=(N // BLKN,),
        in_specs=[nspec, wspec, bspec, wspec, bspec, wspec, bspec],
        out_specs=[nspec, nspec, nspec],
        out_shape=[jax.ShapeDtypeStruct((XROWS, D), jnp.float32),
                   jax.ShapeDtypeStruct((N, D), jnp.float32),
                   jax.ShapeDtypeStruct((N, D), jnp.float32)],
    )(x, we, web, ws, wsb, wr, wrb)


def _sc_logits_body(a_hbm, b_hbm, s_hbm, r_hbm, wa_hbm, logits_hbm, tmax_hbm,
                    sidx, ridx, arows, brows, accbuf, lbuf, wav, mbuf,
                    sem_a, sem_b):
    c = lax.axis_index("c")
    s = lax.axis_index("s")
    wid = c * NS + s
    base = wid * PER_W
    pltpu.sync_copy(wa_hbm, wav)

    def chunk(j, mvec):
        off = base + j * CH
        pltpu.sync_copy(s_hbm.at[pl.ds(off, CH)], sidx.at[0])
        pltpu.sync_copy(r_hbm.at[pl.ds(off, CH)], ridx.at[0])
        cp_a = pltpu.async_copy(a_hbm.at[sidx.at[0]], arows, sem_a)
        cp_b = pltpu.async_copy(b_hbm.at[ridx.at[0]], brows, sem_b)
        cp_a.wait()
        cp_b.wait()

        @pl.loop(0, CH)
        def _edge(e):
            acc = jnp.zeros((L,), jnp.float32)
            for k in range(D // L):
                sl = pl.ds(k * L, L)
                t = arows[e, sl] + brows[e, sl]
                u = 1.0 + jnp.exp(jnp.minimum(t, 30.0))
                u2 = u * u
                z = t * ((u2 - 1.0) / (u2 + 1.0))
                acc = acc + z * wav[sl]
            accbuf[e, :] = acc

        lanes = lax.iota(jnp.int32, L)
        mv = mvec
        for g in range(CH // L):
            rowsum = jnp.zeros((L,), jnp.float32)
            ridx16 = lanes + (g * L)
            for k in range(L):
                col = jnp.full((L,), k, jnp.int32)
                rowsum = rowsum + plsc.load_gather(accbuf, [ridx16, col])
            lbuf[pl.ds(g * L, L)] = rowsum
            mv = jnp.maximum(mv, rowsum)
        pltpu.sync_copy(lbuf, logits_hbm.at[pl.ds(off, CH)])
        return mv

    mvec = lax.fori_loop(0, NCH, chunk, jnp.full((L,), -jnp.inf, jnp.float32))
    mbuf[...] = mvec
    pltpu.sync_copy(mbuf, tmax_hbm.at[pl.ds(wid * L, L)])


def _sc_accum_body(logits_hbm, tmax_hbm, s_hbm, r_hbm, xwe_hbm,
                   num_hbm, den_hbm,
                   sidx, ridx, rows, lbuf, ubuf, tmv, zbuf, den8, shared, sem):
    c = lax.axis_index("c")
    s = lax.axis_index("s")
    wid = c * NS + s
    base = s * PER_W2
    row0 = s * RPT2
    node0 = c * SHALF
    lanes = lax.iota(jnp.int32, L)

    @pl.loop(0, ZR)
    def _zero(i):
        for k in range(D // L):
            zbuf[i, pl.ds(k * L, L)] = jnp.zeros((L,), jnp.float32)

    @pl.loop(0, RPT2 // ZR)
    def _zcopy(m):
        pltpu.sync_copy(zbuf, shared.at[pl.ds(row0 + m * ZR, ZR)])

    @pl.loop(0, (NDEN * N) // L)
    def _zero_den(i):
        den8[pl.ds(i * L, L)] = jnp.zeros((L,), jnp.float32)

    plsc.subcore_barrier()

    pltpu.sync_copy(tmax_hbm, tmv)
    mv = jnp.full((L,), -jnp.inf, jnp.float32)
    for i in range(NW):
        mv = jnp.maximum(mv, tmv[pl.ds(i * L, L)])
    gmax = jnp.max(mv)

    den_masks = [
        jnp.logical_and(jnp.logical_and(lanes >= m * NDEN,
                                        lanes < (m + 1) * NDEN), c == 0)
        for m in range(L // NDEN)
    ]
    tbl_off = (lanes % NDEN) * N

    @pl.loop(0, NCH2)
    def _chunk(j):
        off = base + j * CH
        pltpu.sync_copy(s_hbm.at[pl.ds(off, CH)], sidx.at[0])
        pltpu.sync_copy(r_hbm.at[pl.ds(off, CH)], ridx.at[0])
        pltpu.sync_copy(logits_hbm.at[pl.ds(off, CH)], lbuf)
        pltpu.async_copy(xwe_hbm.at[sidx.at[0]], rows, sem).wait()
        for g in range(CH // L):
            sl = pl.ds(g * L, L)
            u16 = jnp.exp(lbuf[sl] - gmax)
            ubuf[sl] = u16
            r16 = ridx[0, sl]
            didx = tbl_off + r16
            for dm in den_masks:
                plsc.addupdate_scatter(den8, [didx], u16, mask=dm)
            rloc = r16 - node0
            valid = jnp.logical_and(rloc >= 0, rloc < SHALF)
            ridx[0, sl] = jnp.where(valid, rloc, SHALF)

        @pl.loop(0, CH)
        def _edge(e):
            ub = plsc.load_gather(ubuf, [jnp.full((L,), e, jnp.int32)])
            for k in range(D // L):
                sl = pl.ds(k * L, L)
                rows[e, sl] = rows[e, sl] * ub

        pltpu.sync_copy(rows, shared.at[ridx.at[0]], add=True)

    @pl.when(c == 0)
    def _den_out():
        @pl.loop(0, N // L)
        def _red(i):
            sl = pl.ds(i * L, L)
            acc = den8[sl]
            for t in range(1, NDEN):
                acc = acc + den8[pl.ds(t * N + i * L, L)]
            den8[sl] = acc

        pltpu.sync_copy(den8.at[pl.ds(0, N)], den_hbm.at[pl.ds(s * NPAD, N)])

    plsc.subcore_barrier()
    pltpu.sync_copy(shared.at[pl.ds(row0, RPT2)],
                    num_hbm.at[c].at[pl.ds(row0, RPT2)])


def _den_reduce_body(d_ref, o_ref):
    dsum = jnp.sum(d_ref[...], axis=0)
    o_ref[...] = jnp.broadcast_to(dsum[:, None], (BLKC, D))


def _den_reduce(den_partials):
    return pl.pallas_call(
        _den_reduce_body,
        grid=(NPAD // BLKC,),
        in_specs=[pl.BlockSpec((NS, BLKC), lambda i: (0, i))],
        out_specs=pl.BlockSpec((BLKC, D), lambda i: (i, 0)),
        out_shape=jax.ShapeDtypeStruct((NPAD, D), jnp.float32),
    )(den_partials)


def _combine_body(p_ref, d_ref, o_ref):
    num = p_ref[...][0]
    den = d_ref[...]
    o_ref[...] = jnp.where(den > 0.0, num / den, 0.0)


def _combine(num_partials, den_bcast):
    blocks_per_sc = SHALF // BLKO
    return pl.pallas_call(
        _combine_body,
        grid=(N // BLKO,),
        in_specs=[pl.BlockSpec(
                      (1, BLKO, D),
                      lambda i: (i // blocks_per_sc, i % blocks_per_sc, 0)),
                  pl.BlockSpec((BLKO, D), lambda i: (i, 0))],
        out_specs=pl.BlockSpec((BLKO, D), lambda i: (i, 0)),
        out_shape=jax.ShapeDtypeStruct((N, D), jnp.float32),
    )(num_partials, den_bcast)


def kernel(x, senders, receivers, We_k, We_b, Ws_k, Ws_b, Wr_k, Wr_b, Wa_k,
           Wa_b):
    s32 = senders.astype(jnp.int32)
    r32 = receivers.astype(jnp.int32)
    we = We_k.reshape(D, D)
    wr = Wr_k.reshape(D, D)
    web = We_b.reshape(1, D)
    wrb = Wr_b.reshape(1, D)
    wsb = Ws_b.reshape(1, D)
    wa = Wa_k.reshape(D)

    xwe, a, b = _node_transform(x, we, web, Ws_k, wsb, wr, wrb)

    mesh = plsc.VectorSubcoreMesh(core_axis_name="c", subcore_axis_name="s")
    cp = pltpu.CompilerParams()
    if "needs_layout_passes" in pltpu.CompilerParams.__dataclass_fields__:
        cp = dataclasses.replace(cp, needs_layout_passes=False)

    logits_fn = pl.kernel(
        _sc_logits_body,
        out_type=[jax.ShapeDtypeStruct((E,), jnp.float32),
                  jax.ShapeDtypeStruct((NW * L,), jnp.float32)],
        mesh=mesh,
        scratch_types=[
            pltpu.VMEM((1, CH), jnp.int32),
            pltpu.VMEM((1, CH), jnp.int32),
            pltpu.VMEM((CH, D), jnp.float32),
            pltpu.VMEM((CH, D), jnp.float32),
            pltpu.VMEM((CH, L), jnp.float32),
            pltpu.VMEM((CH,), jnp.float32),
            pltpu.VMEM((D,), jnp.float32),
            pltpu.VMEM((L,), jnp.float32),
            pltpu.SemaphoreType.DMA,
            pltpu.SemaphoreType.DMA,
        ],
        compiler_params=cp,
    )
    logits, tmax = logits_fn(a, b, s32, r32, wa)

    accum_fn = pl.kernel(
        _sc_accum_body,
        out_type=[jax.ShapeDtypeStruct((NC, SROWS, D), jnp.float32),
                  jax.ShapeDtypeStruct((NS * NPAD,), jnp.float32)],
        mesh=mesh,
        scratch_types=[
            pltpu.VMEM((1, CH), jnp.int32),
            pltpu.VMEM((1, CH), jnp.int32),
            pltpu.VMEM((CH, D), jnp.float32),
            pltpu.VMEM((CH,), jnp.float32),
            pltpu.VMEM((CH,), jnp.float32),
            pltpu.VMEM((NW * L,), jnp.float32),
            pltpu.VMEM((ZR, D), jnp.float32),
            pltpu.VMEM((NDEN * N,), jnp.float32),
            pltpu.VMEM_SHARED((SROWS, D), jnp.float32),
            pltpu.SemaphoreType.DMA,
        ],
        compiler_params=cp,
    )
    num_partials, den_flat = accum_fn(logits, tmax, s32, r32, xwe)
    den_partials = den_flat.reshape(NS, NPAD)

    return _combine(num_partials, _den_reduce(den_partials))


# speedup vs baseline: 6.4997x; 1.3808x over previous
import dataclasses
import functools

import jax
import jax.numpy as jnp
from jax import lax
from jax.experimental import pallas as pl
from jax.experimental.pallas import tpu as pltpu
from jax.experimental.pallas import tpu_sc as plsc

N = 10000
E = 320000
D = 128
L = 16
NC = 2
NS = 16
NW = NC * NS
PER_W = E // NW
CH = 80
NCH = PER_W // CH
PER_W2 = E // NS
NCH2 = PER_W2 // CH
NPAD = 10240
SHALF = NPAD // NC
SROWS = 5248
RPT2 = SROWS // NS
ZR = 8
NDEN = 4
BLKN = 200
BLKC = 256
BLKO = 80
XROWS = 20400


def _node_transform_body(x_ref, we_ref, web_ref, ws_ref, wsb_ref, wr_ref,
                         wrb_ref, xwe_ref, a_ref, b_ref):
    xb = x_ref[...]
    xwe = lax.dot_general(xb, we_ref[...], (((1,), (0,)), ((), ())),
                          precision=lax.Precision.HIGHEST,
                          preferred_element_type=jnp.float32) + web_ref[...]
    xwe_ref[...] = xwe
    a_ref[...] = lax.dot_general(xwe, ws_ref[...], (((1,), (0,)), ((), ())),
                                 precision=lax.Precision.HIGHEST,
                                 preferred_element_type=jnp.float32) + wsb_ref[...]
    b_ref[...] = lax.dot_general(xb, wr_ref[...], (((1,), (0,)), ((), ())),
                                 precision=lax.Precision.HIGHEST,
                                 preferred_element_type=jnp.float32) + wrb_ref[...]


def _node_transform(x, we, web, ws, wsb, wr, wrb):
    wspec = pl.BlockSpec((D, D), lambda i: (0, 0))
    bspec = pl.BlockSpec((1, D), lambda i: (0, 0))
    nspec = pl.BlockSpec((BLKN, D), lambda i: (i, 0))
    return pl.pallas_call(
        _node_transform_body,
        grid=(N // BLKN,),
        in_specs=[nspec, wspec, bspec, wspec, bspec, wspec, bspec],
        out_specs=[nspec, nspec, nspec],
        out_shape=[jax.ShapeDtypeStruct((XROWS, D), jnp.float32),
                   jax.ShapeDtypeStruct((N, D), jnp.float32),
                   jax.ShapeDtypeStruct((N, D), jnp.float32)],
    )(x, we, web, ws, wsb, wr, wrb)


def _sc_logits_body(a_hbm, b_hbm, s_hbm, r_hbm, wa_hbm, logits_hbm, tmax_hbm,
                    sidx0, sidx1, ridx0, ridx1, arows0, arows1, brows0, brows1,
                    accbuf, lbuf, wav, mbuf,
                    sem_s0, sem_s1, sem_r0, sem_r1,
                    sem_a0, sem_a1, sem_b0, sem_b1):
    c = lax.axis_index("c")
    s = lax.axis_index("s")
    wid = c * NS + s
    base = wid * PER_W
    sidx, ridx = [sidx0, sidx1], [ridx0, ridx1]
    arows, brows = [arows0, arows1], [brows0, brows1]
    sem_s, sem_r = [sem_s0, sem_s1], [sem_r0, sem_r1]
    sem_a, sem_b = [sem_a0, sem_a1], [sem_b0, sem_b1]
    lanes = lax.iota(jnp.int32, L)
    pltpu.sync_copy(wa_hbm, wav)
    mbuf[...] = jnp.full((L,), -jnp.inf, jnp.float32)

    def fire_idx(off, b):
        return (pltpu.async_copy(s_hbm.at[pl.ds(off, CH)], sidx[b].at[0],
                                 sem_s[b]),
                pltpu.async_copy(r_hbm.at[pl.ds(off, CH)], ridx[b].at[0],
                                 sem_r[b]))

    def fire_gather(b):
        return (pltpu.async_copy(a_hbm.at[sidx[b].at[0]], arows[b], sem_a[b]),
                pltpu.async_copy(b_hbm.at[ridx[b].at[0]], brows[b], sem_b[b]))

    def compute(off, b):
        ar, br = arows[b], brows[b]

        @pl.loop(0, CH)
        def _edge(e):
            acc = jnp.zeros((L,), jnp.float32)
            for k in range(D // L):
                sl = pl.ds(k * L, L)
                t = ar[e, sl] + br[e, sl]
                u = 1.0 + jnp.exp(jnp.minimum(t, 30.0))
                u2 = u * u
                z = t * ((u2 - 1.0) / (u2 + 1.0))
                acc = acc + z * wav[sl]
            accbuf[e, :] = acc

        mv = mbuf[...]
        for g in range(CH // L):
            rowsum = jnp.zeros((L,), jnp.float32)
            r16 = lanes + (g * L)
            for k in range(L):
                col = jnp.full((L,), k, jnp.int32)
                rowsum = rowsum + plsc.load_gather(accbuf, [r16, col])
            lbuf[pl.ds(g * L, L)] = rowsum
            mv = jnp.maximum(mv, rowsum)
        mbuf[...] = mv
        pltpu.sync_copy(lbuf, logits_hbm.at[pl.ds(off, CH)])

    @pl.loop(0, NCH // 2)
    def _pair(t):
        off0 = base + (2 * t) * CH
        off1 = off0 + CH
        i0 = fire_idx(off0, 0)
        i1 = fire_idx(off1, 1)
        i0[0].wait()
        i0[1].wait()
        g0 = fire_gather(0)
        i1[0].wait()
        i1[1].wait()
        g1 = fire_gather(1)
        g0[0].wait()
        g0[1].wait()
        compute(off0, 0)
        g1[0].wait()
        g1[1].wait()
        compute(off1, 1)

    off_t = base + (NCH - 1) * CH
    it = fire_idx(off_t, 0)
    it[0].wait()
    it[1].wait()
    gt = fire_gather(0)
    gt[0].wait()
    gt[1].wait()
    compute(off_t, 0)

    pltpu.sync_copy(mbuf, tmax_hbm.at[pl.ds(wid * L, L)])


def _sc_accum_body(logits_hbm, tmax_hbm, s_hbm, r_hbm, xwe_hbm,
                   num_hbm, den_hbm,
                   sidx0, sidx1, ridx0, ridx1, rows0, rows1, lbuf0, lbuf1,
                   ubuf, tmv, zbuf, den8, shared,
                   sem_s0, sem_s1, sem_r0, sem_r1,
                   sem_l0, sem_l1, sem_g0, sem_g1):
    sidx, ridx = [sidx0, sidx1], [ridx0, ridx1]
    rows, lbuf = [rows0, rows1], [lbuf0, lbuf1]
    sem_s, sem_r = [sem_s0, sem_s1], [sem_r0, sem_r1]
    sem_l, sem_g = [sem_l0, sem_l1], [sem_g0, sem_g1]
    c = lax.axis_index("c")
    s = lax.axis_index("s")
    wid = c * NS + s
    base = s * PER_W2
    row0 = s * RPT2
    node0 = c * SHALF
    lanes = lax.iota(jnp.int32, L)

    @pl.loop(0, ZR)
    def _zero(i):
        for k in range(D // L):
            zbuf[i, pl.ds(k * L, L)] = jnp.zeros((L,), jnp.float32)

    @pl.loop(0, RPT2 // ZR)
    def _zcopy(m):
        pltpu.sync_copy(zbuf, shared.at[pl.ds(row0 + m * ZR, ZR)])

    @pl.loop(0, (NDEN * N) // L)
    def _zero_den(i):
        den8[pl.ds(i * L, L)] = jnp.zeros((L,), jnp.float32)

    plsc.subcore_barrier()

    pltpu.sync_copy(tmax_hbm, tmv)
    mv = jnp.full((L,), -jnp.inf, jnp.float32)
    for i in range(NW):
        mv = jnp.maximum(mv, tmv[pl.ds(i * L, L)])
    gmax = jnp.max(mv)

    den_masks = [
        jnp.logical_and(jnp.logical_and(lanes >= m * NDEN,
                                        lanes < (m + 1) * NDEN), c == 0)
        for m in range(L // NDEN)
    ]
    tbl_off = (lanes % NDEN) * N

    def fire_idx(off, b):
        return (pltpu.async_copy(s_hbm.at[pl.ds(off, CH)], sidx[b].at[0],
                                 sem_s[b]),
                pltpu.async_copy(r_hbm.at[pl.ds(off, CH)], ridx[b].at[0],
                                 sem_r[b]),
                pltpu.async_copy(logits_hbm.at[pl.ds(off, CH)], lbuf[b],
                                 sem_l[b]))

    def fire_gather(b):
        return pltpu.async_copy(xwe_hbm.at[sidx[b].at[0]], rows[b], sem_g[b])

    def process(b):
        rw, lb = rows[b], lbuf[b]
        for g in range(CH // L):
            sl = pl.ds(g * L, L)
            u16 = jnp.exp(lb[sl] - gmax)
            ubuf[sl] = u16
            r16 = ridx[b][0, sl]
            didx = tbl_off + r16
            for dm in den_masks:
                plsc.addupdate_scatter(den8, [didx], u16, mask=dm)
            rloc = r16 - node0
            valid = jnp.logical_and(rloc >= 0, rloc < SHALF)
            ridx[b][0, sl] = jnp.where(valid, rloc, SHALF)

        @pl.loop(0, CH)
        def _edge(e):
            ub = plsc.load_gather(ubuf, [jnp.full((L,), e, jnp.int32)])
            for k in range(D // L):
                sl = pl.ds(k * L, L)
                rw[e, sl] = rw[e, sl] * ub

        pltpu.sync_copy(rw, shared.at[ridx[b].at[0]], add=True)

    @pl.loop(0, NCH2 // 2)
    def _pair(t):
        off0 = base + (2 * t) * CH
        off1 = off0 + CH
        i0 = fire_idx(off0, 0)
        i1 = fire_idx(off1, 1)
        i0[0].wait()
        g0 = fire_gather(0)
        i1[0].wait()
        g1 = fire_gather(1)
        i0[1].wait()
        i0[2].wait()
        g0.wait()
        process(0)
        i1[1].wait()
        i1[2].wait()
        g1.wait()
        process(1)

    @pl.when(c == 0)
    def _den_out():
        @pl.loop(0, N // L)
        def _red(i):
            sl = pl.ds(i * L, L)
            acc = den8[sl]
            for t in range(1, NDEN):
                acc = acc + den8[pl.ds(t * N + i * L, L)]
            den8[sl] = acc

        pltpu.sync_copy(den8.at[pl.ds(0, N)], den_hbm.at[pl.ds(s * NPAD, N)])

    plsc.subcore_barrier()
    pltpu.sync_copy(shared.at[pl.ds(row0, RPT2)],
                    num_hbm.at[c].at[pl.ds(row0, RPT2)])


def _den_reduce_body(d_ref, o_ref):
    dsum = jnp.sum(d_ref[...], axis=0)
    o_ref[...] = jnp.broadcast_to(dsum[:, None], (BLKC, D))


def _den_reduce(den_partials):
    return pl.pallas_call(
        _den_reduce_body,
        grid=(NPAD // BLKC,),
        in_specs=[pl.BlockSpec((NS, BLKC), lambda i: (0, i))],
        out_specs=pl.BlockSpec((BLKC, D), lambda i: (i, 0)),
        out_shape=jax.ShapeDtypeStruct((NPAD, D), jnp.float32),
    )(den_partials)


def _combine_body(p_ref, d_ref, o_ref):
    num = p_ref[...][0]
    den = d_ref[...]
    o_ref[...] = jnp.where(den > 0.0, num / den, 0.0)


def _combine(num_partials, den_bcast):
    blocks_per_sc = SHALF // BLKO
    return pl.pallas_call(
        _combine_body,
        grid=(N // BLKO,),
        in_specs=[pl.BlockSpec(
                      (1, BLKO, D),
                      lambda i: (i // blocks_per_sc, i % blocks_per_sc, 0)),
                  pl.BlockSpec((BLKO, D), lambda i: (i, 0))],
        out_specs=pl.BlockSpec((BLKO, D), lambda i: (i, 0)),
        out_shape=jax.ShapeDtypeStruct((N, D), jnp.float32),
    )(num_partials, den_bcast)


def kernel(x, senders, receivers, We_k, We_b, Ws_k, Ws_b, Wr_k, Wr_b, Wa_k,
           Wa_b):
    s32 = senders.astype(jnp.int32)
    r32 = receivers.astype(jnp.int32)
    we = We_k.reshape(D, D)
    wr = Wr_k.reshape(D, D)
    web = We_b.reshape(1, D)
    wrb = Wr_b.reshape(1, D)
    wsb = Ws_b.reshape(1, D)
    wa = Wa_k.reshape(D)

    xwe, a, b = _node_transform(x, we, web, Ws_k, wsb, wr, wrb)

    mesh = plsc.VectorSubcoreMesh(core_axis_name="c", subcore_axis_name="s")
    cp = pltpu.CompilerParams()
    if "needs_layout_passes" in pltpu.CompilerParams.__dataclass_fields__:
        cp = dataclasses.replace(cp, needs_layout_passes=False)

    logits_fn = pl.kernel(
        _sc_logits_body,
        out_type=[jax.ShapeDtypeStruct((E,), jnp.float32),
                  jax.ShapeDtypeStruct((NW * L,), jnp.float32)],
        mesh=mesh,
        scratch_types=(
            [pltpu.VMEM((1, CH), jnp.int32)] * 4 +
            [pltpu.VMEM((CH, D), jnp.float32)] * 4 +
            [pltpu.VMEM((CH, L), jnp.float32),
             pltpu.VMEM((CH,), jnp.float32),
             pltpu.VMEM((D,), jnp.float32),
             pltpu.VMEM((L,), jnp.float32)] +
            [pltpu.SemaphoreType.DMA] * 8
        ),
        compiler_params=cp,
    )
    logits, tmax = logits_fn(a, b, s32, r32, wa)

    accum_fn = pl.kernel(
        _sc_accum_body,
        out_type=[jax.ShapeDtypeStruct((NC, SROWS, D), jnp.float32),
                  jax.ShapeDtypeStruct((NS * NPAD,), jnp.float32)],
        mesh=mesh,
        scratch_types=(
            [pltpu.VMEM((1, CH), jnp.int32)] * 4 +
            [pltpu.VMEM((CH, D), jnp.float32)] * 2 +
            [pltpu.VMEM((CH,), jnp.float32)] * 3 +
            [pltpu.VMEM((NW * L,), jnp.float32),
             pltpu.VMEM((ZR, D), jnp.float32),
             pltpu.VMEM((NDEN * N,), jnp.float32),
             pltpu.VMEM_SHARED((SROWS, D), jnp.float32)] +
            [pltpu.SemaphoreType.DMA] * 8
        ),
        compiler_params=cp,
    )
    num_partials, den_flat = accum_fn(logits, tmax, s32, r32, xwe)
    den_partials = den_flat.reshape(NS, NPAD)

    return _combine(num_partials, _den_reduce(den_partials))


# speedup vs baseline: 7.2838x; 1.1206x over previous
import dataclasses
import functools

import jax
import jax.numpy as jnp
from jax import lax
from jax.experimental import pallas as pl
from jax.experimental.pallas import tpu as pltpu
from jax.experimental.pallas import tpu_sc as plsc

N = 10000
E = 320000
D = 128
L = 16
NC = 2
NS = 16
NW = NC * NS
PER_W = E // NW
CH = 80
NCH = PER_W // CH
PER_W2 = E // NS
NCH2 = PER_W2 // CH
NPAD = 10240
SHALF = NPAD // NC
SROWS = 5248
RPT2 = SROWS // NS
ZR = 8
NDEN = 4
BLKN = 1000
BLKO = 256
XROWS = 20400


def _node_transform_body(x_ref, we_ref, web_ref, ws_ref, wsb_ref, wr_ref,
                         wrb_ref, xwe_ref, a_ref, b_ref):
    xb = x_ref[...]
    xwe = lax.dot_general(xb, we_ref[...], (((1,), (0,)), ((), ())),
                          precision=lax.Precision.HIGHEST,
                          preferred_element_type=jnp.float32) + web_ref[...]
    xwe_ref[...] = xwe
    a_ref[...] = lax.dot_general(xwe, ws_ref[...], (((1,), (0,)), ((), ())),
                                 precision=lax.Precision.HIGHEST,
                                 preferred_element_type=jnp.float32) + wsb_ref[...]
    b_ref[...] = lax.dot_general(xb, wr_ref[...], (((1,), (0,)), ((), ())),
                                 precision=lax.Precision.HIGHEST,
                                 preferred_element_type=jnp.float32) + wrb_ref[...]


def _node_transform(x, we, web, ws, wsb, wr, wrb):
    wspec = pl.BlockSpec((D, D), lambda i: (0, 0))
    bspec = pl.BlockSpec((1, D), lambda i: (0, 0))
    nspec = pl.BlockSpec((BLKN, D), lambda i: (i, 0))
    return pl.pallas_call(
        _node_transform_body,
        grid=(N // BLKN,),
        in_specs=[nspec, wspec, bspec, wspec, bspec, wspec, bspec],
        out_specs=[nspec, nspec, nspec],
        out_shape=[jax.ShapeDtypeStruct((XROWS, D), jnp.float32),
                   jax.ShapeDtypeStruct((N, D), jnp.float32),
                   jax.ShapeDtypeStruct((N, D), jnp.float32)],
    )(x, we, web, ws, wsb, wr, wrb)


def _sc_logits_body(a_hbm, b_hbm, s_hbm, r_hbm, wa_hbm, logits_hbm, tmax_hbm,
                    sidx0, sidx1, ridx0, ridx1, arows0, arows1, brows0, brows1,
                    accbuf, lbuf, wav, mbuf,
                    sem_s0, sem_s1, sem_r0, sem_r1,
                    sem_a0, sem_a1, sem_b0, sem_b1):
    c = lax.axis_index("c")
    s = lax.axis_index("s")
    wid = c * NS + s
    base = wid * PER_W
    sidx, ridx = [sidx0, sidx1], [ridx0, ridx1]
    arows, brows = [arows0, arows1], [brows0, brows1]
    sem_s, sem_r = [sem_s0, sem_s1], [sem_r0, sem_r1]
    sem_a, sem_b = [sem_a0, sem_a1], [sem_b0, sem_b1]
    lanes = lax.iota(jnp.int32, L)
    pltpu.sync_copy(wa_hbm, wav)
    mbuf[...] = jnp.full((L,), -jnp.inf, jnp.float32)

    def fire_idx(off, b):
        return (pltpu.async_copy(s_hbm.at[pl.ds(off, CH)], sidx[b].at[0],
                                 sem_s[b]),
                pltpu.async_copy(r_hbm.at[pl.ds(off, CH)], ridx[b].at[0],
                                 sem_r[b]))

    def fire_gather(b):
        return (pltpu.async_copy(a_hbm.at[sidx[b].at[0]], arows[b], sem_a[b]),
                pltpu.async_copy(b_hbm.at[ridx[b].at[0]], brows[b], sem_b[b]))

    def compute(off, b):
        ar, br = arows[b], brows[b]

        @pl.loop(0, CH)
        def _edge(e):
            acc = jnp.zeros((L,), jnp.float32)
            for k in range(D // L):
                sl = pl.ds(k * L, L)
                t = ar[e, sl] + br[e, sl]
                u = 1.0 + jnp.exp(jnp.minimum(t, 30.0))
                u2 = u * u
                z = t * ((u2 - 1.0) / (u2 + 1.0))
                acc = acc + z * wav[sl]
            accbuf[e, :] = acc

        mv = mbuf[...]
        for g in range(CH // L):
            rowsum = jnp.zeros((L,), jnp.float32)
            r16 = lanes + (g * L)
            for k in range(L):
                col = jnp.full((L,), k, jnp.int32)
                rowsum = rowsum + plsc.load_gather(accbuf, [r16, col])
            lbuf[pl.ds(g * L, L)] = rowsum
            mv = jnp.maximum(mv, rowsum)
        mbuf[...] = mv
        pltpu.sync_copy(lbuf, logits_hbm.at[pl.ds(off, CH)])

    @pl.loop(0, NCH // 2)
    def _pair(t):
        off0 = base + (2 * t) * CH
        off1 = off0 + CH
        i0 = fire_idx(off0, 0)
        i1 = fire_idx(off1, 1)
        i0[0].wait()
        i0[1].wait()
        g0 = fire_gather(0)
        i1[0].wait()
        i1[1].wait()
        g1 = fire_gather(1)
        g0[0].wait()
        g0[1].wait()
        compute(off0, 0)
        g1[0].wait()
        g1[1].wait()
        compute(off1, 1)

    off_t = base + (NCH - 1) * CH
    it = fire_idx(off_t, 0)
    it[0].wait()
    it[1].wait()
    gt = fire_gather(0)
    gt[0].wait()
    gt[1].wait()
    compute(off_t, 0)

    pltpu.sync_copy(mbuf, tmax_hbm.at[pl.ds(wid * L, L)])


def _sc_accum_body(logits_hbm, tmax_hbm, s_hbm, r_hbm, xwe_hbm,
                   num_hbm, den_hbm,
                   sidx0, sidx1, ridx0, ridx1, rows0, rows1, lbuf0, lbuf1,
                   ubuf, tmv, zbuf, den8, shared,
                   sem_s0, sem_s1, sem_r0, sem_r1,
                   sem_l0, sem_l1, sem_g0, sem_g1, sem_c0, sem_c1):
    sidx, ridx = [sidx0, sidx1], [ridx0, ridx1]
    rows, lbuf = [rows0, rows1], [lbuf0, lbuf1]
    sem_s, sem_r = [sem_s0, sem_s1], [sem_r0, sem_r1]
    sem_l, sem_g = [sem_l0, sem_l1], [sem_g0, sem_g1]
    sem_c = [sem_c0, sem_c1]
    c = lax.axis_index("c")
    s = lax.axis_index("s")
    wid = c * NS + s
    base = s * PER_W2
    row0 = s * RPT2
    node0 = c * SHALF
    lanes = lax.iota(jnp.int32, L)

    @pl.loop(0, ZR)
    def _zero(i):
        for k in range(D // L):
            zbuf[i, pl.ds(k * L, L)] = jnp.zeros((L,), jnp.float32)

    @pl.loop(0, RPT2 // ZR)
    def _zcopy(m):
        pltpu.sync_copy(zbuf, shared.at[pl.ds(row0 + m * ZR, ZR)])

    @pl.loop(0, (NDEN * N) // L)
    def _zero_den(i):
        den8[pl.ds(i * L, L)] = jnp.zeros((L,), jnp.float32)

    plsc.subcore_barrier()

    pltpu.sync_copy(tmax_hbm, tmv)
    mv = jnp.full((L,), -jnp.inf, jnp.float32)
    for i in range(NW):
        mv = jnp.maximum(mv, tmv[pl.ds(i * L, L)])
    gmax = jnp.max(mv)

    den_masks = [
        jnp.logical_and(jnp.logical_and(lanes >= m * NDEN,
                                        lanes < (m + 1) * NDEN), c == 0)
        for m in range(L // NDEN)
    ]
    tbl_off = (lanes % NDEN) * N

    def fire_idx(off, b):
        return (pltpu.async_copy(s_hbm.at[pl.ds(off, CH)], sidx[b].at[0],
                                 sem_s[b]),
                pltpu.async_copy(r_hbm.at[pl.ds(off, CH)], ridx[b].at[0],
                                 sem_r[b]),
                pltpu.async_copy(logits_hbm.at[pl.ds(off, CH)], lbuf[b],
                                 sem_l[b]))

    def fire_gather(b):
        return pltpu.async_copy(xwe_hbm.at[sidx[b].at[0]], rows[b], sem_g[b])

    def process(b):
        rw, lb = rows[b], lbuf[b]
        for g in range(CH // L):
            sl = pl.ds(g * L, L)
            u16 = jnp.exp(lb[sl] - gmax)
            ubuf[sl] = u16
            r16 = ridx[b][0, sl]
            didx = tbl_off + r16
            for dm in den_masks:
                plsc.addupdate_scatter(den8, [didx], u16, mask=dm)
            rloc = r16 - node0
            valid = jnp.logical_and(rloc >= 0, rloc < SHALF)
            ridx[b][0, sl] = jnp.where(valid, rloc, SHALF)

        @pl.loop(0, CH)
        def _edge(e):
            ub = plsc.load_gather(ubuf, [jnp.full((L,), e, jnp.int32)])
            for k in range(D // L):
                sl = pl.ds(k * L, L)
                rw[e, sl] = rw[e, sl] * ub

        pltpu.async_copy(rw, shared.at[ridx[b].at[0]], sem_c[b], add=True)

    def wait_scatter(b):
        pltpu.make_async_copy(rows[b], shared.at[ridx[b].at[0]],
                              sem_c[b]).wait()

    @pl.loop(0, NCH2 // 2)
    def _pair(t):
        @pl.when(t > 0)
        def _drain():
            wait_scatter(0)
            wait_scatter(1)

        off0 = base + (2 * t) * CH
        off1 = off0 + CH
        i0 = fire_idx(off0, 0)
        i1 = fire_idx(off1, 1)
        i0[0].wait()
        g0 = fire_gather(0)
        i1[0].wait()
        g1 = fire_gather(1)
        i0[1].wait()
        i0[2].wait()
        g0.wait()
        process(0)
        i1[1].wait()
        i1[2].wait()
        g1.wait()
        process(1)

    wait_scatter(0)
    wait_scatter(1)

    @pl.when(c == 0)
    def _den_out():
        @pl.loop(0, N // L)
        def _red(i):
            sl = pl.ds(i * L, L)
            acc = den8[sl]
            for t in range(1, NDEN):
                acc = acc + den8[pl.ds(t * N + i * L, L)]
            den8[sl] = acc

        pltpu.sync_copy(den8.at[pl.ds(0, N)], den_hbm.at[pl.ds(s * NPAD, N)])

    plsc.subcore_barrier()
    pltpu.sync_copy(shared.at[pl.ds(row0, RPT2)],
                    num_hbm.at[c].at[pl.ds(row0, RPT2)])


def _combine_body(p_ref, d_ref, o_ref):
    num = p_ref[...][0]
    den = jnp.sum(d_ref[...], axis=0)[:, None]
    o_ref[...] = jnp.where(den > 0.0, num / den, 0.0)


def _combine(num_partials, den_partials):
    blocks_per_sc = SHALF // BLKO
    return pl.pallas_call(
        _combine_body,
        grid=(NPAD // BLKO,),
        in_specs=[pl.BlockSpec(
                      (1, BLKO, D),
                      lambda i: (i // blocks_per_sc, i % blocks_per_sc, 0)),
                  pl.BlockSpec((NS, BLKO), lambda i: (0, i))],
        out_specs=pl.BlockSpec((BLKO, D), lambda i: (i, 0)),
        out_shape=jax.ShapeDtypeStruct((NPAD, D), jnp.float32),
    )(num_partials, den_partials)


def kernel(x, senders, receivers, We_k, We_b, Ws_k, Ws_b, Wr_k, Wr_b, Wa_k,
           Wa_b):
    s32 = senders.astype(jnp.int32)
    r32 = receivers.astype(jnp.int32)
    we = We_k.reshape(D, D)
    wr = Wr_k.reshape(D, D)
    web = We_b.reshape(1, D)
    wrb = Wr_b.reshape(1, D)
    wsb = Ws_b.reshape(1, D)
    wa = Wa_k.reshape(D)

    xwe, a, b = _node_transform(x, we, web, Ws_k, wsb, wr, wrb)

    mesh = plsc.VectorSubcoreMesh(core_axis_name="c", subcore_axis_name="s")
    cp = pltpu.CompilerParams()
    if "needs_layout_passes" in pltpu.CompilerParams.__dataclass_fields__:
        cp = dataclasses.replace(cp, needs_layout_passes=False)

    logits_fn = pl.kernel(
        _sc_logits_body,
        out_type=[jax.ShapeDtypeStruct((E,), jnp.float32),
                  jax.ShapeDtypeStruct((NW * L,), jnp.float32)],
        mesh=mesh,
        scratch_types=(
            [pltpu.VMEM((1, CH), jnp.int32)] * 4 +
            [pltpu.VMEM((CH, D), jnp.float32)] * 4 +
            [pltpu.VMEM((CH, L), jnp.float32),
             pltpu.VMEM((CH,), jnp.float32),
             pltpu.VMEM((D,), jnp.float32),
             pltpu.VMEM((L,), jnp.float32)] +
            [pltpu.SemaphoreType.DMA] * 8
        ),
        compiler_params=cp,
    )
    logits, tmax = logits_fn(a, b, s32, r32, wa)

    accum_fn = pl.kernel(
        _sc_accum_body,
        out_type=[jax.ShapeDtypeStruct((NC, SROWS, D), jnp.float32),
                  jax.ShapeDtypeStruct((NS * NPAD,), jnp.float32)],
        mesh=mesh,
        scratch_types=(
            [pltpu.VMEM((1, CH), jnp.int32)] * 4 +
            [pltpu.VMEM((CH, D), jnp.float32)] * 2 +
            [pltpu.VMEM((CH,), jnp.float32)] * 3 +
            [pltpu.VMEM((NW * L,), jnp.float32),
             pltpu.VMEM((ZR, D), jnp.float32),
             pltpu.VMEM((NDEN * N,), jnp.float32),
             pltpu.VMEM_SHARED((SROWS, D), jnp.float32)] +
            [pltpu.SemaphoreType.DMA] * 10
        ),
        compiler_params=cp,
    )
    num_partials, den_flat = accum_fn(logits, tmax, s32, r32, xwe)
    den_partials = den_flat.reshape(NS, NPAD)

    return _combine(num_partials, den_partials)[:N]


# speedup vs baseline: 7.9415x; 1.0903x over previous
import dataclasses
import functools

import jax
import jax.numpy as jnp
from jax import lax
from jax.experimental import pallas as pl
from jax.experimental.pallas import tpu as pltpu
from jax.experimental.pallas import tpu_sc as plsc

N = 10000
E = 320000
D = 128
L = 16
NC = 2
NS = 16
NW = NC * NS
PER_W = E // NW
CH = 80
NCH = PER_W // CH
PER_W2 = E // NS
NCH2 = PER_W2 // CH
NPAD = 10240
SHALF = NPAD // NC
SROWS = 5248
RPT2 = SROWS // NS
ZR = 8
NDEN = 4
BLKN = 1000
BLKO = 256
XROWS = 20400


def _node_transform_body(x_ref, we_ref, web_ref, ws_ref, wsb_ref, wr_ref,
                         wrb_ref, xwe_ref, a_ref, b_ref):
    xb = x_ref[...]
    xwe = lax.dot_general(xb, we_ref[...], (((1,), (0,)), ((), ())),
                          precision=lax.Precision.HIGHEST,
                          preferred_element_type=jnp.float32) + web_ref[...]
    xwe_ref[...] = xwe
    a_ref[...] = lax.dot_general(xwe, ws_ref[...], (((1,), (0,)), ((), ())),
                                 precision=lax.Precision.HIGHEST,
                                 preferred_element_type=jnp.float32) + wsb_ref[...]
    b_ref[...] = lax.dot_general(xb, wr_ref[...], (((1,), (0,)), ((), ())),
                                 precision=lax.Precision.HIGHEST,
                                 preferred_element_type=jnp.float32) + wrb_ref[...]


def _node_transform(x, we, web, ws, wsb, wr, wrb):
    wspec = pl.BlockSpec((D, D), lambda i: (0, 0))
    bspec = pl.BlockSpec((1, D), lambda i: (0, 0))
    nspec = pl.BlockSpec((BLKN, D), lambda i: (i, 0))
    return pl.pallas_call(
        _node_transform_body,
        grid=(N // BLKN,),
        in_specs=[nspec, wspec, bspec, wspec, bspec, wspec, bspec],
        out_specs=[nspec, nspec, nspec],
        out_shape=[jax.ShapeDtypeStruct((XROWS, D), jnp.float32),
                   jax.ShapeDtypeStruct((N, D), jnp.float32),
                   jax.ShapeDtypeStruct((N, D), jnp.float32)],
    )(x, we, web, ws, wsb, wr, wrb)


def _sc_logits_body(a_hbm, b_hbm, s_hbm, r_hbm, wa_hbm, logits_hbm, tmax_hbm,
                    sidx0, sidx1, ridx0, ridx1, arows0, arows1, brows0, brows1,
                    accbuf, lbuf, wav, mbuf,
                    sem_s0, sem_s1, sem_r0, sem_r1,
                    sem_a0, sem_a1, sem_b0, sem_b1):
    c = lax.axis_index("c")
    s = lax.axis_index("s")
    wid = c * NS + s
    base = wid * PER_W
    sidx, ridx = [sidx0, sidx1], [ridx0, ridx1]
    arows, brows = [arows0, arows1], [brows0, brows1]
    sem_s, sem_r = [sem_s0, sem_s1], [sem_r0, sem_r1]
    sem_a, sem_b = [sem_a0, sem_a1], [sem_b0, sem_b1]
    lanes = lax.iota(jnp.int32, L)
    pltpu.sync_copy(wa_hbm, wav)
    mbuf[...] = jnp.full((L,), -jnp.inf, jnp.float32)

    def fire_idx(off, b):
        return (pltpu.async_copy(s_hbm.at[pl.ds(off, CH)], sidx[b].at[0],
                                 sem_s[b]),
                pltpu.async_copy(r_hbm.at[pl.ds(off, CH)], ridx[b].at[0],
                                 sem_r[b]))

    def fire_gather(b):
        return (pltpu.async_copy(a_hbm.at[sidx[b].at[0]], arows[b], sem_a[b]),
                pltpu.async_copy(b_hbm.at[ridx[b].at[0]], brows[b], sem_b[b]))

    def compute(off, b):
        ar, br = arows[b], brows[b]

        @plsc.parallel_loop(0, CH, unroll=4)
        def _edge(e):
            acc = jnp.zeros((L,), jnp.float32)
            for k in range(D // L):
                sl = pl.ds(k * L, L)
                t = ar[e, sl] + br[e, sl]
                u = 1.0 + jnp.exp(jnp.minimum(t, 30.0))
                u2 = u * u
                z = t * ((u2 - 1.0) / (u2 + 1.0))
                acc = acc + z * wav[sl]
            accbuf[e, :] = acc

        mv = mbuf[...]
        for g in range(CH // L):
            rowsum = jnp.zeros((L,), jnp.float32)
            r16 = lanes + (g * L)
            for k in range(L):
                col = jnp.full((L,), k, jnp.int32)
                rowsum = rowsum + plsc.load_gather(accbuf, [r16, col])
            lbuf[pl.ds(g * L, L)] = rowsum
            mv = jnp.maximum(mv, rowsum)
        mbuf[...] = mv
        pltpu.sync_copy(lbuf, logits_hbm.at[pl.ds(off, CH)])

    @pl.loop(0, NCH // 2)
    def _pair(t):
        off0 = base + (2 * t) * CH
        off1 = off0 + CH
        i0 = fire_idx(off0, 0)
        i1 = fire_idx(off1, 1)
        i0[0].wait()
        i0[1].wait()
        g0 = fire_gather(0)
        i1[0].wait()
        i1[1].wait()
        g1 = fire_gather(1)
        g0[0].wait()
        g0[1].wait()
        compute(off0, 0)
        g1[0].wait()
        g1[1].wait()
        compute(off1, 1)

    off_t = base + (NCH - 1) * CH
    it = fire_idx(off_t, 0)
    it[0].wait()
    it[1].wait()
    gt = fire_gather(0)
    gt[0].wait()
    gt[1].wait()
    compute(off_t, 0)

    pltpu.sync_copy(mbuf, tmax_hbm.at[pl.ds(wid * L, L)])


def _sc_accum_body(logits_hbm, tmax_hbm, s_hbm, r_hbm, xwe_hbm,
                   num_hbm, den_hbm,
                   sidx0, sidx1, ridx0, ridx1, rows0, rows1, lbuf0, lbuf1,
                   ubuf, tmv, zbuf, den8, shared,
                   sem_s0, sem_s1, sem_r0, sem_r1,
                   sem_l0, sem_l1, sem_g0, sem_g1, sem_c0, sem_c1):
    sidx, ridx = [sidx0, sidx1], [ridx0, ridx1]
    rows, lbuf = [rows0, rows1], [lbuf0, lbuf1]
    sem_s, sem_r = [sem_s0, sem_s1], [sem_r0, sem_r1]
    sem_l, sem_g = [sem_l0, sem_l1], [sem_g0, sem_g1]
    sem_c = [sem_c0, sem_c1]
    c = lax.axis_index("c")
    s = lax.axis_index("s")
    wid = c * NS + s
    base = s * PER_W2
    row0 = s * RPT2
    node0 = c * SHALF
    lanes = lax.iota(jnp.int32, L)

    @pl.loop(0, ZR)
    def _zero(i):
        for k in range(D // L):
            zbuf[i, pl.ds(k * L, L)] = jnp.zeros((L,), jnp.float32)

    @pl.loop(0, RPT2 // ZR)
    def _zcopy(m):
        pltpu.sync_copy(zbuf, shared.at[pl.ds(row0 + m * ZR, ZR)])

    @pl.loop(0, (NDEN * N) // L)
    def _zero_den(i):
        den8[pl.ds(i * L, L)] = jnp.zeros((L,), jnp.float32)

    plsc.subcore_barrier()

    pltpu.sync_copy(tmax_hbm, tmv)
    mv = jnp.full((L,), -jnp.inf, jnp.float32)
    for i in range(NW):
        mv = jnp.maximum(mv, tmv[pl.ds(i * L, L)])
    gmax = jnp.max(mv)

    den_masks = [
        jnp.logical_and(jnp.logical_and(lanes >= m * NDEN,
                                        lanes < (m + 1) * NDEN), c == 0)
        for m in range(L // NDEN)
    ]
    tbl_off = (lanes % NDEN) * N

    def fire_idx(off, b):
        return (pltpu.async_copy(s_hbm.at[pl.ds(off, CH)], sidx[b].at[0],
                                 sem_s[b]),
                pltpu.async_copy(r_hbm.at[pl.ds(off, CH)], ridx[b].at[0],
                                 sem_r[b]),
                pltpu.async_copy(logits_hbm.at[pl.ds(off, CH)], lbuf[b],
                                 sem_l[b]))

    def fire_gather(b):
        return pltpu.async_copy(xwe_hbm.at[sidx[b].at[0]], rows[b], sem_g[b])

    def process(b):
        rw, lb = rows[b], lbuf[b]
        for g in range(CH // L):
            sl = pl.ds(g * L, L)
            u16 = jnp.exp(lb[sl] - gmax)
            ubuf[sl] = u16
            r16 = ridx[b][0, sl]
            didx = tbl_off + r16
            for dm in den_masks:
                plsc.addupdate_scatter(den8, [didx], u16, mask=dm)
            rloc = r16 - node0
            valid = jnp.logical_and(rloc >= 0, rloc < SHALF)
            ridx[b][0, sl] = jnp.where(valid, rloc, SHALF)

        @plsc.parallel_loop(0, CH, unroll=4)
        def _edge(e):
            ub = plsc.load_gather(ubuf, [jnp.full((L,), e, jnp.int32)])
            for k in range(D // L):
                sl = pl.ds(k * L, L)
                rw[e, sl] = rw[e, sl] * ub

        pltpu.async_copy(rw, shared.at[ridx[b].at[0]], sem_c[b], add=True)

    def wait_scatter(b):
        pltpu.make_async_copy(rows[b], shared.at[ridx[b].at[0]],
                              sem_c[b]).wait()

    @pl.loop(0, NCH2 // 2)
    def _pair(t):
        @pl.when(t > 0)
        def _drain():
            wait_scatter(0)
            wait_scatter(1)

        off0 = base + (2 * t) * CH
        off1 = off0 + CH
        i0 = fire_idx(off0, 0)
        i1 = fire_idx(off1, 1)
        i0[0].wait()
        g0 = fire_gather(0)
        i1[0].wait()
        g1 = fire_gather(1)
        i0[1].wait()
        i0[2].wait()
        g0.wait()
        process(0)
        i1[1].wait()
        i1[2].wait()
        g1.wait()
        process(1)

    wait_scatter(0)
    wait_scatter(1)

    @pl.when(c == 0)
    def _den_out():
        @pl.loop(0, N // L)
        def _red(i):
            sl = pl.ds(i * L, L)
            acc = den8[sl]
            for t in range(1, NDEN):
                acc = acc + den8[pl.ds(t * N + i * L, L)]
            den8[sl] = acc

        pltpu.sync_copy(den8.at[pl.ds(0, N)], den_hbm.at[pl.ds(s * NPAD, N)])

    plsc.subcore_barrier()
    pltpu.sync_copy(shared.at[pl.ds(row0, RPT2)],
                    num_hbm.at[c].at[pl.ds(row0, RPT2)])


def _combine_body(p_ref, d_ref, o_ref):
    num = p_ref[...][0]
    den = jnp.sum(d_ref[...], axis=0)[:, None]
    o_ref[...] = jnp.where(den > 0.0, num / den, 0.0)


def _combine(num_partials, den_partials):
    blocks_per_sc = SHALF // BLKO
    return pl.pallas_call(
        _combine_body,
        grid=(NPAD // BLKO,),
        in_specs=[pl.BlockSpec(
                      (1, BLKO, D),
                      lambda i: (i // blocks_per_sc, i % blocks_per_sc, 0)),
                  pl.BlockSpec((NS, BLKO), lambda i: (0, i))],
        out_specs=pl.BlockSpec((BLKO, D), lambda i: (i, 0)),
        out_shape=jax.ShapeDtypeStruct((NPAD, D), jnp.float32),
    )(num_partials, den_partials)


def kernel(x, senders, receivers, We_k, We_b, Ws_k, Ws_b, Wr_k, Wr_b, Wa_k,
           Wa_b):
    s32 = senders.astype(jnp.int32)
    r32 = receivers.astype(jnp.int32)
    we = We_k.reshape(D, D)
    wr = Wr_k.reshape(D, D)
    web = We_b.reshape(1, D)
    wrb = Wr_b.reshape(1, D)
    wsb = Ws_b.reshape(1, D)
    wa = Wa_k.reshape(D)

    xwe, a, b = _node_transform(x, we, web, Ws_k, wsb, wr, wrb)

    mesh = plsc.VectorSubcoreMesh(core_axis_name="c", subcore_axis_name="s")
    cp = pltpu.CompilerParams()
    if "needs_layout_passes" in pltpu.CompilerParams.__dataclass_fields__:
        cp = dataclasses.replace(cp, needs_layout_passes=False)

    logits_fn = pl.kernel(
        _sc_logits_body,
        out_type=[jax.ShapeDtypeStruct((E,), jnp.float32),
                  jax.ShapeDtypeStruct((NW * L,), jnp.float32)],
        mesh=mesh,
        scratch_types=(
            [pltpu.VMEM((1, CH), jnp.int32)] * 4 +
            [pltpu.VMEM((CH, D), jnp.float32)] * 4 +
            [pltpu.VMEM((CH, L), jnp.float32),
             pltpu.VMEM((CH,), jnp.float32),
             pltpu.VMEM((D,), jnp.float32),
             pltpu.VMEM((L,), jnp.float32)] +
            [pltpu.SemaphoreType.DMA] * 8
        ),
        compiler_params=cp,
    )
    logits, tmax = logits_fn(a, b, s32, r32, wa)

    accum_fn = pl.kernel(
        _sc_accum_body,
        out_type=[jax.ShapeDtypeStruct((NC, SROWS, D), jnp.float32),
                  jax.ShapeDtypeStruct((NS * NPAD,), jnp.float32)],
        mesh=mesh,
        scratch_types=(
            [pltpu.VMEM((1, CH), jnp.int32)] * 4 +
            [pltpu.VMEM((CH, D), jnp.float32)] * 2 +
            [pltpu.VMEM((CH,), jnp.float32)] * 3 +
            [pltpu.VMEM((NW * L,), jnp.float32),
             pltpu.VMEM((ZR, D), jnp.float32),
             pltpu.VMEM((NDEN * N,), jnp.float32),
             pltpu.VMEM_SHARED((SROWS, D), jnp.float32)] +
            [pltpu.SemaphoreType.DMA] * 10
        ),
        compiler_params=cp,
    )
    num_partials, den_flat = accum_fn(logits, tmax, s32, r32, xwe)
    den_partials = den_flat.reshape(NS, NPAD)

    return _combine(num_partials, den_partials)[:N]


# speedup vs baseline: 8.0484x; 1.0135x over previous
import dataclasses
import functools

import jax
import jax.numpy as jnp
from jax import lax
from jax.experimental import pallas as pl
from jax.experimental.pallas import tpu as pltpu
from jax.experimental.pallas import tpu_sc as plsc

N = 10000
E = 320000
D = 128
L = 16
NC = 2
NS = 16
NW = NC * NS
PER_W = E // NW
CH = 80
NCH = PER_W // CH
PER_W2 = E // NS
NCH2 = PER_W2 // CH
NPAD = 10240
SHALF = NPAD // NC
SROWS = 5248
RPT2 = SROWS // NS
ZR = 8
NDEN = 4
BLKN = 1000
BLKO = 256
XROWS = 20400


def _node_transform_body(x_ref, we_ref, web_ref, ws_ref, wsb_ref, wr_ref,
                         wrb_ref, xwe_ref, a_ref, b_ref):
    xb = x_ref[...]
    xwe = lax.dot_general(xb, we_ref[...], (((1,), (0,)), ((), ())),
                          precision=lax.Precision.HIGHEST,
                          preferred_element_type=jnp.float32) + web_ref[...]
    xwe_ref[...] = xwe
    a_ref[...] = lax.dot_general(xwe, ws_ref[...], (((1,), (0,)), ((), ())),
                                 precision=lax.Precision.HIGHEST,
                                 preferred_element_type=jnp.float32) + wsb_ref[...]
    b_ref[...] = lax.dot_general(xb, wr_ref[...], (((1,), (0,)), ((), ())),
                                 precision=lax.Precision.HIGHEST,
                                 preferred_element_type=jnp.float32) + wrb_ref[...]


def _node_transform(x, we, web, ws, wsb, wr, wrb):
    wspec = pl.BlockSpec((D, D), lambda i: (0, 0))
    bspec = pl.BlockSpec((1, D), lambda i: (0, 0))
    nspec = pl.BlockSpec((BLKN, D), lambda i: (i, 0))
    return pl.pallas_call(
        _node_transform_body,
        grid=(N // BLKN,),
        in_specs=[nspec, wspec, bspec, wspec, bspec, wspec, bspec],
        out_specs=[nspec, nspec, nspec],
        out_shape=[jax.ShapeDtypeStruct((XROWS, D), jnp.float32),
                   jax.ShapeDtypeStruct((N, D), jnp.float32),
                   jax.ShapeDtypeStruct((N, D), jnp.float32)],
    )(x, we, web, ws, wsb, wr, wrb)


def _sc_logits_body(a_hbm, b_hbm, s_hbm, r_hbm, wa_hbm, logits_hbm, tmax_hbm,
                    sidx0, sidx1, ridx0, ridx1, arows0, arows1, brows0, brows1,
                    accbuf, lbuf, wav, mbuf,
                    sem_s0, sem_s1, sem_r0, sem_r1,
                    sem_a0, sem_a1, sem_b0, sem_b1):
    c = lax.axis_index("c")
    s = lax.axis_index("s")
    wid = c * NS + s
    base = wid * PER_W
    sidx, ridx = [sidx0, sidx1], [ridx0, ridx1]
    arows, brows = [arows0, arows1], [brows0, brows1]
    sem_s, sem_r = [sem_s0, sem_s1], [sem_r0, sem_r1]
    sem_a, sem_b = [sem_a0, sem_a1], [sem_b0, sem_b1]
    lanes = lax.iota(jnp.int32, L)
    pltpu.sync_copy(wa_hbm, wav)
    mbuf[...] = jnp.full((L,), -jnp.inf, jnp.float32)

    def fire_idx(off, b):
        return (pltpu.async_copy(s_hbm.at[pl.ds(off, CH)], sidx[b].at[0],
                                 sem_s[b]),
                pltpu.async_copy(r_hbm.at[pl.ds(off, CH)], ridx[b].at[0],
                                 sem_r[b]))

    def fire_gather(b):
        return (pltpu.async_copy(a_hbm.at[sidx[b].at[0]], arows[b], sem_a[b]),
                pltpu.async_copy(b_hbm.at[ridx[b].at[0]], brows[b], sem_b[b]))

    def compute(off, b):
        ar, br = arows[b], brows[b]

        @plsc.parallel_loop(0, CH, unroll=4)
        def _edge(e):
            acc = jnp.zeros((L,), jnp.float32)
            for k in range(D // L):
                sl = pl.ds(k * L, L)
                t = ar[e, sl] + br[e, sl]
                u = 1.0 + jnp.exp(jnp.minimum(t, 30.0))
                u2 = u * u
                z = t * ((u2 - 1.0) / (u2 + 1.0))
                acc = acc + z * wav[sl]
            accbuf[e, :] = acc

        mv = mbuf[...]
        for g in range(CH // L):
            rowsum = jnp.zeros((L,), jnp.float32)
            r16 = lanes + (g * L)
            for k in range(L):
                col = jnp.full((L,), k, jnp.int32)
                rowsum = rowsum + plsc.load_gather(accbuf, [r16, col])
            lbuf[pl.ds(g * L, L)] = rowsum
            mv = jnp.maximum(mv, rowsum)
        mbuf[...] = mv
        pltpu.sync_copy(lbuf, logits_hbm.at[pl.ds(off, CH)])

    @pl.loop(0, NCH // 2)
    def _pair(t):
        off0 = base + (2 * t) * CH
        off1 = off0 + CH
        i0 = fire_idx(off0, 0)
        i1 = fire_idx(off1, 1)
        i0[0].wait()
        i0[1].wait()
        g0 = fire_gather(0)
        i1[0].wait()
        i1[1].wait()
        g1 = fire_gather(1)
        g0[0].wait()
        g0[1].wait()
        compute(off0, 0)
        g1[0].wait()
        g1[1].wait()
        compute(off1, 1)

    off_t = base + (NCH - 1) * CH
    it = fire_idx(off_t, 0)
    it[0].wait()
    it[1].wait()
    gt = fire_gather(0)
    gt[0].wait()
    gt[1].wait()
    compute(off_t, 0)

    pltpu.sync_copy(mbuf, tmax_hbm.at[pl.ds(wid * L, L)])


def _sc_accum_body(logits_hbm, tmax_hbm, s_hbm, r_hbm, xwe_hbm,
                   num_hbm, den_hbm,
                   sidx0, sidx1, ridx0, ridx1, rows0, rows1, lbuf0, lbuf1,
                   ubuf, tmv, zbuf, den8, shared,
                   sem_s0, sem_s1, sem_r0, sem_r1,
                   sem_l0, sem_l1, sem_g0, sem_g1, sem_c0, sem_c1):
    sidx, ridx = [sidx0, sidx1], [ridx0, ridx1]
    rows, lbuf = [rows0, rows1], [lbuf0, lbuf1]
    sem_s, sem_r = [sem_s0, sem_s1], [sem_r0, sem_r1]
    sem_l, sem_g = [sem_l0, sem_l1], [sem_g0, sem_g1]
    sem_c = [sem_c0, sem_c1]
    c = lax.axis_index("c")
    s = lax.axis_index("s")
    wid = c * NS + s
    base = s * PER_W2
    row0 = s * RPT2
    node0 = c * SHALF
    lanes = lax.iota(jnp.int32, L)

    @pl.loop(0, ZR)
    def _zero(i):
        for k in range(D // L):
            zbuf[i, pl.ds(k * L, L)] = jnp.zeros((L,), jnp.float32)

    @pl.loop(0, RPT2 // ZR)
    def _zcopy(m):
        pltpu.sync_copy(zbuf, shared.at[pl.ds(row0 + m * ZR, ZR)])

    @plsc.parallel_loop(0, (NDEN * N) // L, unroll=8)
    def _zero_den(i):
        den8[pl.ds(i * L, L)] = jnp.zeros((L,), jnp.float32)

    plsc.subcore_barrier()

    pltpu.sync_copy(tmax_hbm, tmv)
    mv = jnp.full((L,), -jnp.inf, jnp.float32)
    for i in range(NW):
        mv = jnp.maximum(mv, tmv[pl.ds(i * L, L)])
    gmax = jnp.max(mv)

    den_masks = [
        jnp.logical_and(jnp.logical_and(lanes >= m * NDEN,
                                        lanes < (m + 1) * NDEN), c == 0)
        for m in range(L // NDEN)
    ]
    tbl_off = (lanes % NDEN) * N

    def fire_idx(off, b):
        return (pltpu.async_copy(s_hbm.at[pl.ds(off, CH)], sidx[b].at[0],
                                 sem_s[b]),
                pltpu.async_copy(r_hbm.at[pl.ds(off, CH)], ridx[b].at[0],
                                 sem_r[b]),
                pltpu.async_copy(logits_hbm.at[pl.ds(off, CH)], lbuf[b],
                                 sem_l[b]))

    def fire_gather(b):
        return pltpu.async_copy(xwe_hbm.at[sidx[b].at[0]], rows[b], sem_g[b])

    def process(b):
        rw, lb = rows[b], lbuf[b]
        for g in range(CH // L):
            sl = pl.ds(g * L, L)
            u16 = jnp.exp(lb[sl] - gmax)
            ubuf[sl] = u16
            r16 = ridx[b][0, sl]
            didx = tbl_off + r16
            for dm in den_masks:
                plsc.addupdate_scatter(den8, [didx], u16, mask=dm)
            rloc = r16 - node0
            valid = jnp.logical_and(rloc >= 0, rloc < SHALF)
            ridx[b][0, sl] = jnp.where(valid, rloc, SHALF)

        @plsc.parallel_loop(0, CH, unroll=4)
        def _edge(e):
            ub = plsc.load_gather(ubuf, [jnp.full((L,), e, jnp.int32)])
            for k in range(D // L):
                sl = pl.ds(k * L, L)
                rw[e, sl] = rw[e, sl] * ub

        pltpu.async_copy(rw, shared.at[ridx[b].at[0]], sem_c[b], add=True)

    def wait_scatter(b):
        pltpu.make_async_copy(rows[b], shared.at[ridx[b].at[0]],
                              sem_c[b]).wait()

    @pl.loop(0, NCH2 // 2)
    def _pair(t):
        @pl.when(t > 0)
        def _drain():
            wait_scatter(0)
            wait_scatter(1)

        off0 = base + (2 * t) * CH
        off1 = off0 + CH
        i0 = fire_idx(off0, 0)
        i1 = fire_idx(off1, 1)
        i0[0].wait()
        g0 = fire_gather(0)
        i1[0].wait()
        g1 = fire_gather(1)
        i0[1].wait()
        i0[2].wait()
        g0.wait()
        process(0)
        i1[1].wait()
        i1[2].wait()
        g1.wait()
        process(1)

    wait_scatter(0)
    wait_scatter(1)

    @pl.when(c == 0)
    def _den_out():
        @plsc.parallel_loop(0, N // L, unroll=4)
        def _red(i):
            sl = pl.ds(i * L, L)
            acc = den8[sl]
            for t in range(1, NDEN):
                acc = acc + den8[pl.ds(t * N + i * L, L)]
            den8[sl] = acc

        pltpu.sync_copy(den8.at[pl.ds(0, N)], den_hbm.at[pl.ds(s * NPAD, N)])

    plsc.subcore_barrier()
    pltpu.sync_copy(shared.at[pl.ds(row0, RPT2)],
                    num_hbm.at[c].at[pl.ds(row0, RPT2)])


def _combine_body(p_ref, d_ref, o_ref):
    num = p_ref[...][0]
    den = jnp.sum(d_ref[...], axis=0)[:, None]
    o_ref[...] = jnp.where(den > 0.0, num / den, 0.0)


def _combine(num_partials, den_partials):
    blocks_per_sc = SHALF // BLKO
    return pl.pallas_call(
        _combine_body,
        grid=(NPAD // BLKO,),
        in_specs=[pl.BlockSpec(
                      (1, BLKO, D),
                      lambda i: (i // blocks_per_sc, i % blocks_per_sc, 0)),
                  pl.BlockSpec((NS, BLKO), lambda i: (0, i))],
        out_specs=pl.BlockSpec((BLKO, D), lambda i: (i, 0)),
        out_shape=jax.ShapeDtypeStruct((NPAD, D), jnp.float32),
    )(num_partials, den_partials)


def kernel(x, senders, receivers, We_k, We_b, Ws_k, Ws_b, Wr_k, Wr_b, Wa_k,
           Wa_b):
    s32 = senders.astype(jnp.int32)
    r32 = receivers.astype(jnp.int32)
    we = We_k.reshape(D, D)
    wr = Wr_k.reshape(D, D)
    web = We_b.reshape(1, D)
    wrb = Wr_b.reshape(1, D)
    wsb = Ws_b.reshape(1, D)
    wa = Wa_k.reshape(D)

    xwe, a, b = _node_transform(x, we, web, Ws_k, wsb, wr, wrb)

    mesh = plsc.VectorSubcoreMesh(core_axis_name="c", subcore_axis_name="s")
    cp = pltpu.CompilerParams()
    if "needs_layout_passes" in pltpu.CompilerParams.__dataclass_fields__:
        cp = dataclasses.replace(cp, needs_layout_passes=False)

    logits_fn = pl.kernel(
        _sc_logits_body,
        out_type=[jax.ShapeDtypeStruct((E,), jnp.float32),
                  jax.ShapeDtypeStruct((NW * L,), jnp.float32)],
        mesh=mesh,
        scratch_types=(
            [pltpu.VMEM((1, CH), jnp.int32)] * 4 +
            [pltpu.VMEM((CH, D), jnp.float32)] * 4 +
            [pltpu.VMEM((CH, L), jnp.float32),
             pltpu.VMEM((CH,), jnp.float32),
             pltpu.VMEM((D,), jnp.float32),
             pltpu.VMEM((L,), jnp.float32)] +
            [pltpu.SemaphoreType.DMA] * 8
        ),
        compiler_params=cp,
    )
    logits, tmax = logits_fn(a, b, s32, r32, wa)

    accum_fn = pl.kernel(
        _sc_accum_body,
        out_type=[jax.ShapeDtypeStruct((NC, SROWS, D), jnp.float32),
                  jax.ShapeDtypeStruct((NS * NPAD,), jnp.float32)],
        mesh=mesh,
        scratch_types=(
            [pltpu.VMEM((1, CH), jnp.int32)] * 4 +
            [pltpu.VMEM((CH, D), jnp.float32)] * 2 +
            [pltpu.VMEM((CH,), jnp.float32)] * 3 +
            [pltpu.VMEM((NW * L,), jnp.float32),
             pltpu.VMEM((ZR, D), jnp.float32),
             pltpu.VMEM((NDEN * N,), jnp.float32),
             pltpu.VMEM_SHARED((SROWS, D), jnp.float32)] +
            [pltpu.SemaphoreType.DMA] * 10
        ),
        compiler_params=cp,
    )
    num_partials, den_flat = accum_fn(logits, tmax, s32, r32, xwe)
    den_partials = den_flat.reshape(NS, NPAD)

    return _combine(num_partials, den_partials)[:N]


# speedup vs baseline: 8.2039x; 1.0193x over previous
import dataclasses
import functools

import jax
import jax.numpy as jnp
from jax import lax
from jax.experimental import pallas as pl
from jax.experimental.pallas import tpu as pltpu
from jax.experimental.pallas import tpu_sc as plsc

N = 10000
E = 320000
D = 128
L = 16
NC = 2
NS = 16
NW = NC * NS
PER_W = E // NW
CH = 80
NCH = PER_W // CH
PER_W2 = E // NS
NCH2 = PER_W2 // CH
NPAD = 10240
SHALF = NPAD // NC
SROWS = 5248
RPT2 = SROWS // NS
ZR = 8
NDEN = 4
BLKN = 1000
BLKO = 256
XROWS = 20400


def _node_transform_body(x_ref, we_ref, web_ref, ws_ref, wsb_ref, wr_ref,
                         wrb_ref, xwe_ref, a_ref, b_ref):
    xb = x_ref[...]
    xwe = lax.dot_general(xb, we_ref[...], (((1,), (0,)), ((), ())),
                          precision=lax.Precision.HIGHEST,
                          preferred_element_type=jnp.float32) + web_ref[...]
    xwe_ref[...] = xwe
    a_ref[...] = lax.dot_general(xwe, ws_ref[...], (((1,), (0,)), ((), ())),
                                 precision=lax.Precision.HIGHEST,
                                 preferred_element_type=jnp.float32) + wsb_ref[...]
    b_ref[...] = lax.dot_general(xb, wr_ref[...], (((1,), (0,)), ((), ())),
                                 precision=lax.Precision.HIGHEST,
                                 preferred_element_type=jnp.float32) + wrb_ref[...]


def _node_transform(x, we, web, ws, wsb, wr, wrb):
    wspec = pl.BlockSpec((D, D), lambda i: (0, 0))
    bspec = pl.BlockSpec((1, D), lambda i: (0, 0))
    nspec = pl.BlockSpec((BLKN, D), lambda i: (i, 0))
    return pl.pallas_call(
        _node_transform_body,
        grid=(N // BLKN,),
        in_specs=[nspec, wspec, bspec, wspec, bspec, wspec, bspec],
        out_specs=[nspec, nspec, nspec],
        out_shape=[jax.ShapeDtypeStruct((XROWS, D), jnp.float32),
                   jax.ShapeDtypeStruct((N, D), jnp.float32),
                   jax.ShapeDtypeStruct((N, D), jnp.float32)],
    )(x, we, web, ws, wsb, wr, wrb)


def _sc_logits_body(a_hbm, b_hbm, s_hbm, r_hbm, wa_hbm, logits_hbm, tmax_hbm,
                    sidx0, sidx1, ridx0, ridx1, arows, brows0, brows1,
                    accbuf, lbuf, wav, mbuf, sha,
                    sem_s0, sem_s1, sem_r0, sem_r1,
                    sem_a, sem_b0, sem_b1):
    c = lax.axis_index("c")
    s = lax.axis_index("s")
    wid = c * NS + s
    base = wid * PER_W
    sidx, ridx = [sidx0, sidx1], [ridx0, ridx1]
    brows = [brows0, brows1]
    sem_s, sem_r = [sem_s0, sem_s1], [sem_r0, sem_r1]
    sem_b = [sem_b0, sem_b1]
    lanes = lax.iota(jnp.int32, L)
    pltpu.sync_copy(wa_hbm, wav)
    mbuf[...] = jnp.full((L,), -jnp.inf, jnp.float32)

    @pl.when(s != NS - 1)
    def _stage_a():
        pltpu.sync_copy(a_hbm.at[pl.ds(s * 624, 624)],
                        sha.at[pl.ds(s * 624, 624)])

    @pl.when(s == NS - 1)
    def _stage_a_last():
        pltpu.sync_copy(a_hbm.at[pl.ds(9360, N - 9360)],
                        sha.at[pl.ds(9360, N - 9360)])

    plsc.subcore_barrier()

    def fire_idx(off, b):
        return (pltpu.async_copy(s_hbm.at[pl.ds(off, CH)], sidx[b].at[0],
                                 sem_s[b]),
                pltpu.async_copy(r_hbm.at[pl.ds(off, CH)], ridx[b].at[0],
                                 sem_r[b]))

    def fire_a(b):
        return pltpu.async_copy(sha.at[sidx[b].at[0]], arows, sem_a)

    def fire_b(b):
        return pltpu.async_copy(b_hbm.at[ridx[b].at[0]], brows[b], sem_b[b])

    def compute(off, b):
        ar, br = arows, brows[b]

        @plsc.parallel_loop(0, CH, unroll=4)
        def _edge(e):
            acc = jnp.zeros((L,), jnp.float32)
            for k in range(D // L):
                sl = pl.ds(k * L, L)
                t = ar[e, sl] + br[e, sl]
                u = 1.0 + jnp.exp(jnp.minimum(t, 30.0))
                u2 = u * u
                z = t * ((u2 - 1.0) / (u2 + 1.0))
                acc = acc + z * wav[sl]
            accbuf[e, :] = acc

        mv = mbuf[...]
        for g in range(CH // L):
            rowsum = jnp.zeros((L,), jnp.float32)
            r16 = lanes + (g * L)
            for k in range(L):
                col = jnp.full((L,), k, jnp.int32)
                rowsum = rowsum + plsc.load_gather(accbuf, [r16, col])
            lbuf[pl.ds(g * L, L)] = rowsum
            mv = jnp.maximum(mv, rowsum)
        mbuf[...] = mv
        pltpu.sync_copy(lbuf, logits_hbm.at[pl.ds(off, CH)])

    @pl.loop(0, NCH // 2)
    def _pair(t):
        off0 = base + (2 * t) * CH
        off1 = off0 + CH
        i0 = fire_idx(off0, 0)
        i1 = fire_idx(off1, 1)
        i0[0].wait()
        i0[1].wait()
        ga0 = fire_a(0)
        gb0 = fire_b(0)
        i1[0].wait()
        i1[1].wait()
        gb1 = fire_b(1)
        ga0.wait()
        gb0.wait()
        compute(off0, 0)
        ga1 = fire_a(1)
        ga1.wait()
        gb1.wait()
        compute(off1, 1)

    off_t = base + (NCH - 1) * CH
    it = fire_idx(off_t, 0)
    it[0].wait()
    it[1].wait()
    gat = fire_a(0)
    gbt = fire_b(0)
    gat.wait()
    gbt.wait()
    compute(off_t, 0)

    pltpu.sync_copy(mbuf, tmax_hbm.at[pl.ds(wid * L, L)])


def _sc_accum_body(logits_hbm, tmax_hbm, s_hbm, r_hbm, xwe_hbm,
                   num_hbm, den_hbm,
                   sidx0, sidx1, ridx0, ridx1, rows0, rows1, lbuf0, lbuf1,
                   ubuf, tmv, zbuf, den8, shared,
                   sem_s0, sem_s1, sem_r0, sem_r1,
                   sem_l0, sem_l1, sem_g0, sem_g1, sem_c0, sem_c1):
    sidx, ridx = [sidx0, sidx1], [ridx0, ridx1]
    rows, lbuf = [rows0, rows1], [lbuf0, lbuf1]
    sem_s, sem_r = [sem_s0, sem_s1], [sem_r0, sem_r1]
    sem_l, sem_g = [sem_l0, sem_l1], [sem_g0, sem_g1]
    sem_c = [sem_c0, sem_c1]
    c = lax.axis_index("c")
    s = lax.axis_index("s")
    wid = c * NS + s
    base = s * PER_W2
    row0 = s * RPT2
    node0 = c * SHALF
    lanes = lax.iota(jnp.int32, L)

    @pl.loop(0, ZR)
    def _zero(i):
        for k in range(D // L):
            zbuf[i, pl.ds(k * L, L)] = jnp.zeros((L,), jnp.float32)

    @pl.loop(0, RPT2 // ZR)
    def _zcopy(m):
        pltpu.sync_copy(zbuf, shared.at[pl.ds(row0 + m * ZR, ZR)])

    @plsc.parallel_loop(0, (NDEN * N) // L, unroll=8)
    def _zero_den(i):
        den8[pl.ds(i * L, L)] = jnp.zeros((L,), jnp.float32)

    plsc.subcore_barrier()

    pltpu.sync_copy(tmax_hbm, tmv)
    mv = jnp.full((L,), -jnp.inf, jnp.float32)
    for i in range(NW):
        mv = jnp.maximum(mv, tmv[pl.ds(i * L, L)])
    gmax = jnp.max(mv)

    den_masks = [
        jnp.logical_and(jnp.logical_and(lanes >= m * NDEN,
                                        lanes < (m + 1) * NDEN), c == 0)
        for m in range(L // NDEN)
    ]
    tbl_off = (lanes % NDEN) * N

    def fire_idx(off, b):
        return (pltpu.async_copy(s_hbm.at[pl.ds(off, CH)], sidx[b].at[0],
                                 sem_s[b]),
                pltpu.async_copy(r_hbm.at[pl.ds(off, CH)], ridx[b].at[0],
                                 sem_r[b]),
                pltpu.async_copy(logits_hbm.at[pl.ds(off, CH)], lbuf[b],
                                 sem_l[b]))

    def fire_gather(b):
        return pltpu.async_copy(xwe_hbm.at[sidx[b].at[0]], rows[b], sem_g[b])

    def process(b):
        rw, lb = rows[b], lbuf[b]
        for g in range(CH // L):
            sl = pl.ds(g * L, L)
            u16 = jnp.exp(lb[sl] - gmax)
            ubuf[sl] = u16
            r16 = ridx[b][0, sl]
            didx = tbl_off + r16
            for dm in den_masks:
                plsc.addupdate_scatter(den8, [didx], u16, mask=dm)
            rloc = r16 - node0
            valid = jnp.logical_and(rloc >= 0, rloc < SHALF)
            ridx[b][0, sl] = jnp.where(valid, rloc, SHALF)

        @plsc.parallel_loop(0, CH, unroll=4)
        def _edge(e):
            ub = plsc.load_gather(ubuf, [jnp.full((L,), e, jnp.int32)])
            for k in range(D // L):
                sl = pl.ds(k * L, L)
                rw[e, sl] = rw[e, sl] * ub

        pltpu.async_copy(rw, shared.at[ridx[b].at[0]], sem_c[b], add=True)

    def wait_scatter(b):
        pltpu.make_async_copy(rows[b], shared.at[ridx[b].at[0]],
                              sem_c[b]).wait()

    @pl.loop(0, NCH2 // 2)
    def _pair(t):
        @pl.when(t > 0)
        def _drain():
            wait_scatter(0)
            wait_scatter(1)

        off0 = base + (2 * t) * CH
        off1 = off0 + CH
        i0 = fire_idx(off0, 0)
        i1 = fire_idx(off1, 1)
        i0[0].wait()
        g0 = fire_gather(0)
        i1[0].wait()
        g1 = fire_gather(1)
        i0[1].wait()
        i0[2].wait()
        g0.wait()
        process(0)
        i1[1].wait()
        i1[2].wait()
        g1.wait()
        process(1)

    wait_scatter(0)
    wait_scatter(1)

    @pl.when(c == 0)
    def _den_out():
        @plsc.parallel_loop(0, N // L, unroll=4)
        def _red(i):
            sl = pl.ds(i * L, L)
            acc = den8[sl]
            for t in range(1, NDEN):
                acc = acc + den8[pl.ds(t * N + i * L, L)]
            den8[sl] = acc

        pltpu.sync_copy(den8.at[pl.ds(0, N)], den_hbm.at[pl.ds(s * NPAD, N)])

    plsc.subcore_barrier()
    pltpu.sync_copy(shared.at[pl.ds(row0, RPT2)],
                    num_hbm.at[c].at[pl.ds(row0, RPT2)])


def _combine_body(p_ref, d_ref, o_ref):
    num = p_ref[...][0]
    den = jnp.sum(d_ref[...], axis=0)[:, None]
    o_ref[...] = jnp.where(den > 0.0, num / den, 0.0)


def _combine(num_partials, den_partials):
    blocks_per_sc = SHALF // BLKO
    return pl.pallas_call(
        _combine_body,
        grid=(NPAD // BLKO,),
        in_specs=[pl.BlockSpec(
                      (1, BLKO, D),
                      lambda i: (i // blocks_per_sc, i % blocks_per_sc, 0)),
                  pl.BlockSpec((NS, BLKO), lambda i: (0, i))],
        out_specs=pl.BlockSpec((BLKO, D), lambda i: (i, 0)),
        out_shape=jax.ShapeDtypeStruct((NPAD, D), jnp.float32),
    )(num_partials, den_partials)


def kernel(x, senders, receivers, We_k, We_b, Ws_k, Ws_b, Wr_k, Wr_b, Wa_k,
           Wa_b):
    s32 = senders.astype(jnp.int32)
    r32 = receivers.astype(jnp.int32)
    we = We_k.reshape(D, D)
    wr = Wr_k.reshape(D, D)
    web = We_b.reshape(1, D)
    wrb = Wr_b.reshape(1, D)
    wsb = Ws_b.reshape(1, D)
    wa = Wa_k.reshape(D)

    xwe, a, b = _node_transform(x, we, web, Ws_k, wsb, wr, wrb)

    mesh = plsc.VectorSubcoreMesh(core_axis_name="c", subcore_axis_name="s")
    cp = pltpu.CompilerParams()
    if "needs_layout_passes" in pltpu.CompilerParams.__dataclass_fields__:
        cp = dataclasses.replace(cp, needs_layout_passes=False)

    logits_fn = pl.kernel(
        _sc_logits_body,
        out_type=[jax.ShapeDtypeStruct((E,), jnp.float32),
                  jax.ShapeDtypeStruct((NW * L,), jnp.float32)],
        mesh=mesh,
        scratch_types=(
            [pltpu.VMEM((1, CH), jnp.int32)] * 4 +
            [pltpu.VMEM((CH, D), jnp.float32)] * 3 +
            [pltpu.VMEM((CH, L), jnp.float32),
             pltpu.VMEM((CH,), jnp.float32),
             pltpu.VMEM((D,), jnp.float32),
             pltpu.VMEM((L,), jnp.float32),
             pltpu.VMEM_SHARED((N, D), jnp.float32)] +
            [pltpu.SemaphoreType.DMA] * 7
        ),
        compiler_params=cp,
    )
    logits, tmax = logits_fn(a, b, s32, r32, wa)

    accum_fn = pl.kernel(
        _sc_accum_body,
        out_type=[jax.ShapeDtypeStruct((NC, SROWS, D), jnp.float32),
                  jax.ShapeDtypeStruct((NS * NPAD,), jnp.float32)],
        mesh=mesh,
        scratch_types=(
            [pltpu.VMEM((1, CH), jnp.int32)] * 4 +
            [pltpu.VMEM((CH, D), jnp.float32)] * 2 +
            [pltpu.VMEM((CH,), jnp.float32)] * 3 +
            [pltpu.VMEM((NW * L,), jnp.float32),
             pltpu.VMEM((ZR, D), jnp.float32),
             pltpu.VMEM((NDEN * N,), jnp.float32),
             pltpu.VMEM_SHARED((SROWS, D), jnp.float32)] +
            [pltpu.SemaphoreType.DMA] * 10
        ),
        compiler_params=cp,
    )
    num_partials, den_flat = accum_fn(logits, tmax, s32, r32, xwe)
    den_partials = den_flat.reshape(NS, NPAD)

    return _combine(num_partials, den_partials)[:N]
